# baseline (speedup 1.0000x reference)
import jax
import jax.numpy as jnp
from jax.experimental import pallas as pl
from jax.experimental.pallas import tpu as pltpu

LEVELS = 1024
TIMESTAMPS = 2048
DIM = 10000
SIGNAL_MIN = -3.0
SIGNAL_MAX = 3.0


def _value_to_index(value, low, high, num):
    idx = jnp.round((value - low) / (high - low) * (num - 1))
    return jnp.clip(idx, 0, num - 1).astype(jnp.int32)


def _body(xi, yi, zi, ti, tx, ty, tz, tt, out):
    i = pl.program_id(0)
    prod = tx[0] * ty[0] * tz[0] * tt[0]

    @pl.when(i == 0)
    def _init():
        out[...] = prod

    @pl.when(i > 0)
    def _acc():
        out[...] += prod

    @pl.when(i == pl.num_programs(0) - 1)
    def _finish():
        out[...] = jnp.sin(jnp.sign(out[...]))


def kernel(input, table_x, table_y, table_z, table_t):
    x = jnp.clip(input[:, 1], SIGNAL_MIN, SIGNAL_MAX)
    y = jnp.clip(input[:, 2], SIGNAL_MIN, SIGNAL_MAX)
    z = jnp.clip(input[:, 3], SIGNAL_MIN, SIGNAL_MAX)
    xi = _value_to_index(x, SIGNAL_MIN, SIGNAL_MAX, LEVELS)
    yi = _value_to_index(y, SIGNAL_MIN, SIGNAL_MAX, LEVELS)
    zi = _value_to_index(z, SIGNAL_MIN, SIGNAL_MAX, LEVELS)
    ti = _value_to_index(input[:, 0], 0.0, float(TIMESTAMPS), TIMESTAMPS)

    tx3 = table_x.reshape(LEVELS, 1, DIM)
    ty3 = table_y.reshape(LEVELS, 1, DIM)
    tz3 = table_z.reshape(LEVELS, 1, DIM)
    tt3 = table_t.reshape(TIMESTAMPS, 1, DIM)

    out = pl.pallas_call(
        _body,
        grid_spec=pltpu.PrefetchScalarGridSpec(
            num_scalar_prefetch=4,
            grid=(TIMESTAMPS,),
            in_specs=[
                pl.BlockSpec((1, 1, DIM), lambda i, xi, yi, zi, ti: (xi[i], 0, 0)),
                pl.BlockSpec((1, 1, DIM), lambda i, xi, yi, zi, ti: (yi[i], 0, 0)),
                pl.BlockSpec((1, 1, DIM), lambda i, xi, yi, zi, ti: (zi[i], 0, 0)),
                pl.BlockSpec((1, 1, DIM), lambda i, xi, yi, zi, ti: (ti[i], 0, 0)),
            ],
            out_specs=pl.BlockSpec((1, DIM), lambda i, *_: (0, 0)),
        ),
        out_shape=jax.ShapeDtypeStruct((1, DIM), jnp.float32),
    )(xi, yi, zi, ti, tx3, ty3, tz3, tt3)
    return out[0]


# speedup vs baseline: 2.1286x; 2.1286x over previous
import jax
import jax.numpy as jnp
from jax.experimental import pallas as pl
from jax.experimental.pallas import tpu as pltpu

LEVELS = 1024
TIMESTAMPS = 2048
DIM = 10000
SIGNAL_MIN = -3.0
SIGNAL_MAX = 3.0
K = 8


def _value_to_index(value, low, high, num):
    idx = jnp.round((value - low) / (high - low) * (num - 1))
    return jnp.clip(idx, 0, num - 1).astype(jnp.int32)


def _body(xi, yi, zi, ti, *refs):
    out = refs[-1]
    rows = refs[:-1]
    i = pl.program_id(0)
    acc = jnp.zeros((1, DIM), jnp.float32)
    for j in range(K):
        acc += rows[j][0] * rows[K + j][0] * rows[2 * K + j][0] * rows[3 * K + j][0]

    @pl.when(i == 0)
    def _init():
        out[...] = acc

    @pl.when(i > 0)
    def _acc():
        out[...] += acc

    @pl.when(i == pl.num_programs(0) - 1)
    def _finish():
        out[...] = jnp.sin(jnp.sign(out[...]))


def kernel(input, table_x, table_y, table_z, table_t):
    x = jnp.clip(input[:, 1], SIGNAL_MIN, SIGNAL_MAX)
    y = jnp.clip(input[:, 2], SIGNAL_MIN, SIGNAL_MAX)
    z = jnp.clip(input[:, 3], SIGNAL_MIN, SIGNAL_MAX)
    xi = _value_to_index(x, SIGNAL_MIN, SIGNAL_MAX, LEVELS)
    yi = _value_to_index(y, SIGNAL_MIN, SIGNAL_MAX, LEVELS)
    zi = _value_to_index(z, SIGNAL_MIN, SIGNAL_MAX, LEVELS)
    ti = _value_to_index(input[:, 0], 0.0, float(TIMESTAMPS), TIMESTAMPS)

    tx3 = table_x.reshape(LEVELS, 1, DIM)
    ty3 = table_y.reshape(LEVELS, 1, DIM)
    tz3 = table_z.reshape(LEVELS, 1, DIM)
    tt3 = table_t.reshape(TIMESTAMPS, 1, DIM)

    def spec(idx_pos, j):
        def imap(i, xi, yi, zi, ti):
            sel = (xi, yi, zi, ti)[idx_pos]
            return (sel[i * K + j], 0, 0)

        return pl.BlockSpec((1, 1, DIM), imap)

    in_specs = [spec(p, j) for p in range(4) for j in range(K)]

    out = pl.pallas_call(
        _body,
        grid_spec=pltpu.PrefetchScalarGridSpec(
            num_scalar_prefetch=4,
            grid=(TIMESTAMPS // K,),
            in_specs=in_specs,
            out_specs=pl.BlockSpec((1, DIM), lambda i, *_: (0, 0)),
        ),
        out_shape=jax.ShapeDtypeStruct((1, DIM), jnp.float32),
    )(xi, yi, zi, ti, *([tx3] * K + [ty3] * K + [tz3] * K + [tt3] * K))
    return out[0]


# speedup vs baseline: 3.7463x; 1.7600x over previous
import jax
import jax.numpy as jnp
from jax import lax
from jax.experimental import pallas as pl
from jax.experimental.pallas import tpu as pltpu
from jax.experimental.pallas import tpu_sc as plsc

LEVELS = 1024
TIMESTAMPS = 2048
DIM = 10000
SIGNAL_MIN = -3.0
SIGNAL_MAX = 3.0

NC = 2
NS = 16
LANES = 16
NW = NC * NS
DMAIN = 9984
NWIDE = 14
WWIDE = 384
WNARR = 256
G = 16
NGRP = TIMESTAMPS // G
SIN1 = 0.8414709848078965


def _value_to_index(value, low, high, num):
    idx = jnp.round((value - low) / (high - low) * (num - 1))
    return jnp.clip(idx, 0, num - 1).astype(jnp.int32)


def _sc_body(idx_hbm, tx, ty, tz, tt, out_hbm, idx_v, bufs, acc_v, sems):
    cid = lax.axis_index("c")
    s = lax.axis_index("s")
    w = s * NC + cid
    tables = (tx, ty, tz, tt)

    pltpu.sync_copy(idx_hbm, idx_v)

    def run(col0, width):
        nch = width // LANES

        def zf(c, carry):
            acc_v[pl.ds(c * LANES, LANES)] = jnp.zeros((LANES,), jnp.float32)
            return carry

        lax.fori_loop(0, nch, zf, 0)

        def issue(g, slot):
            off = pl.multiple_of(g * G, 8)
            for t in range(4):
                pltpu.async_copy(
                    tables[t].at[idx_v.at[t, pl.ds(off, G)],
                                 pl.ds(col0, width)],
                    bufs.at[slot, t, :, pl.ds(0, width)],
                    sems.at[slot, t],
                )

        def drain(g, slot):
            off = pl.multiple_of(g * G, 8)
            for t in range(4):
                pltpu.make_async_copy(
                    tables[t].at[idx_v.at[t, pl.ds(off, G)],
                                 pl.ds(col0, width)],
                    bufs.at[slot, t, :, pl.ds(0, width)],
                    sems.at[slot, t],
                ).wait()

        issue(0, 0)

        def gloop(gg, carry):
            for b in range(2):
                g = gg * 2 + b

                @pl.when(g + 1 < NGRP)
                def _prefetch():
                    issue(g + 1, 1 - b)

                drain(g, b)

                def cf(c, carry2):
                    sl = pl.ds(c * LANES, LANES)
                    a = acc_v[sl]
                    for j in range(G):
                        p = bufs[b, 0, j, sl] * bufs[b, 1, j, sl]
                        p = p * bufs[b, 2, j, sl]
                        p = p * bufs[b, 3, j, sl]
                        a = a + p
                    acc_v[sl] = a
                    return carry2

                lax.fori_loop(0, nch, cf, 0)
            return carry

        lax.fori_loop(0, NGRP // 2, gloop, 0)

        def ff(c, carry):
            sl = pl.ds(c * LANES, LANES)
            acc_v[sl] = jnp.sign(acc_v[sl]) * jnp.float32(SIN1)
            return carry

        lax.fori_loop(0, nch, ff, 0)
        pltpu.sync_copy(acc_v.at[pl.ds(0, width)], out_hbm.at[pl.ds(col0, width)])

    @pl.when(w < NWIDE)
    def _wide():
        run(pl.multiple_of(w * WWIDE, 128), WWIDE)

    @pl.when(w >= NWIDE)
    def _narrow():
        run(pl.multiple_of(NWIDE * WWIDE - NWIDE * WNARR + w * WNARR, 128),
            WNARR)


def _tail_body(xi, yi, zi, ti, tx, ty, tz, tt, o_ref):
    rows_l = lax.broadcasted_iota(jnp.int32, (1, LEVELS), 1)
    rows_t = lax.broadcasted_iota(jnp.int32, (1, TIMESTAMPS), 1)

    def emb(idx_ref, tab_ref, rows):
        oh = (idx_ref[...].reshape(TIMESTAMPS, 1) == rows).astype(jnp.float32)
        return jnp.dot(oh, tab_ref[...], preferred_element_type=jnp.float32)

    ex = emb(xi, tx, rows_l)
    ey = emb(yi, ty, rows_l)
    ez = emb(zi, tz, rows_l)
    et = emb(ti, tt, rows_t)
    total = jnp.sum(ex * ey * ez * et, axis=0)
    o_ref[...] = jnp.sin(jnp.sign(total))[None]


@jax.jit
def _sc_encode(idx, table_x, table_y, table_z, table_t):
    mesh = plsc.VectorSubcoreMesh(
        core_axis_name="c", subcore_axis_name="s", num_cores=NC, num_subcores=NS
    )
    main = pl.kernel(
        _sc_body,
        out_type=jax.ShapeDtypeStruct((DMAIN,), jnp.float32),
        mesh=mesh,
        scratch_types=[
            pltpu.VMEM((4, TIMESTAMPS), jnp.int32),
            pltpu.VMEM((2, 4, G, WWIDE), jnp.float32),
            pltpu.VMEM((WWIDE,), jnp.float32),
            pltpu.SemaphoreType.DMA((2, 4)),
        ],
    )(idx, table_x, table_y, table_z, table_t)

    tail = pl.pallas_call(
        _tail_body,
        grid=(1,),
        in_specs=[
            pl.BlockSpec((TIMESTAMPS,), lambda i: (0,)),
            pl.BlockSpec((TIMESTAMPS,), lambda i: (0,)),
            pl.BlockSpec((TIMESTAMPS,), lambda i: (0,)),
            pl.BlockSpec((TIMESTAMPS,), lambda i: (0,)),
            pl.BlockSpec((LEVELS, 128), lambda i: (0, DMAIN // 128)),
            pl.BlockSpec((LEVELS, 128), lambda i: (0, DMAIN // 128)),
            pl.BlockSpec((LEVELS, 128), lambda i: (0, DMAIN // 128)),
            pl.BlockSpec((TIMESTAMPS, 128), lambda i: (0, DMAIN // 128)),
        ],
        out_specs=pl.BlockSpec((1, 128), lambda i: (0, 0)),
        out_shape=jax.ShapeDtypeStruct((1, 128), jnp.float32),
    )(idx[0], idx[1], idx[2], idx[3], table_x, table_y, table_z, table_t)

    return jnp.concatenate([main, tail[0, : DIM - DMAIN]])


def kernel(input, table_x, table_y, table_z, table_t):
    x = jnp.clip(input[:, 1], SIGNAL_MIN, SIGNAL_MAX)
    y = jnp.clip(input[:, 2], SIGNAL_MIN, SIGNAL_MAX)
    z = jnp.clip(input[:, 3], SIGNAL_MIN, SIGNAL_MAX)
    xi = _value_to_index(x, SIGNAL_MIN, SIGNAL_MAX, LEVELS)
    yi = _value_to_index(y, SIGNAL_MIN, SIGNAL_MAX, LEVELS)
    zi = _value_to_index(z, SIGNAL_MIN, SIGNAL_MAX, LEVELS)
    ti = _value_to_index(input[:, 0], 0.0, float(TIMESTAMPS), TIMESTAMPS)
    idx = jnp.stack([xi, yi, zi, ti], axis=0)
    return _sc_encode(idx, table_x, table_y, table_z, table_t)
